# baseline (speedup 1.0000x reference)
import functools

import jax
import jax.numpy as jnp
from jax import lax
from jax.experimental import pallas as pl
from jax.experimental.pallas import tpu as pltpu
from jax.experimental.pallas import tpu_sc as plsc

_NW = 32
_L = 16


def _col_gather(tt_flat, it, n_rows):
    d, b = it.shape
    cols_per_w = d // _NW
    mesh = plsc.VectorSubcoreMesh(core_axis_name="c", subcore_axis_name="s")

    @functools.partial(
        pl.kernel,
        mesh=mesh,
        out_type=jax.ShapeDtypeStruct((d, b), jnp.float32),
        scratch_types=[
            pltpu.VMEM((b,), jnp.int32),
            pltpu.VMEM((b,), jnp.float32),
            pltpu.SemaphoreType.DMA,
        ],
    )
    def k(tt_hbm, it_hbm, out_hbm, idx_v, out_v, sem):
        wid = lax.axis_index("s") * 2 + lax.axis_index("c")
        for j in range(cols_per_w):
            c = wid * cols_per_w + j
            pltpu.sync_copy(it_hbm.at[c], idx_v)
            col_base = c * n_rows

            def body(v, _):
                off = v * _L
                idx_v[pl.ds(off, _L)] = idx_v[pl.ds(off, _L)] + col_base
                return 0

            lax.fori_loop(0, b // _L, body, 0)
            pltpu.async_copy(tt_hbm.at[idx_v], out_v, sem).wait()
            pltpu.sync_copy(out_v, out_hbm.at[c])

    return k(tt_flat, it)


def kernel(input_tensor, indices):
    n_rows, d = input_tensor.shape
    tt_flat = input_tensor.T.reshape(-1)
    out_t = _col_gather(tt_flat, indices.astype(jnp.int32).T, n_rows)
    return out_t.T


# speedup vs baseline: 7.6494x; 7.6494x over previous
import functools

import jax
import jax.numpy as jnp
from jax import lax
from jax.experimental import pallas as pl
from jax.experimental.pallas import tpu as pltpu
from jax.experimental.pallas import tpu_sc as plsc

_NW = 32
_L = 16


def _col_gather(tt_flat, it, n_rows):
    d, b = it.shape
    cols_per_w = d // _NW
    mesh = plsc.VectorSubcoreMesh(core_axis_name="c", subcore_axis_name="s")

    @functools.partial(
        pl.kernel,
        mesh=mesh,
        out_type=jax.ShapeDtypeStruct((d, b), jnp.float32),
        scratch_types=[
            pltpu.VMEM((b,), jnp.int32),
            pltpu.VMEM((b,), jnp.float32),
            pltpu.SemaphoreType.DMA,
        ],
    )
    def k(tt_hbm, it_hbm, out_hbm, idx_v, out_v, sem):
        wid = lax.axis_index("s") * 2 + lax.axis_index("c")
        for j in range(cols_per_w):
            c = wid * cols_per_w + j
            pltpu.sync_copy(it_hbm.at[c], idx_v)

            def body(v, _):
                off = v * _L
                idx_v[pl.ds(off, _L)] = (idx_v[pl.ds(off, _L)] * d) + c
                return 0

            lax.fori_loop(0, b // _L, body, 0)
            pltpu.async_copy(tt_hbm.at[idx_v], out_v, sem).wait()
            pltpu.sync_copy(out_v, out_hbm.at[c])

    return k(tt_flat, it)


def kernel(input_tensor, indices):
    n_rows, d = input_tensor.shape
    flat = input_tensor.reshape(-1)
    out_t = _col_gather(flat, indices.astype(jnp.int32).T, n_rows)
    return out_t.T


# speedup vs baseline: 19.9017x; 2.6017x over previous
import functools

import jax
import jax.numpy as jnp
from jax import lax
from jax.experimental import pallas as pl
from jax.experimental.pallas import tpu as pltpu
from jax.experimental.pallas import tpu_sc as plsc

_NW = 32
_L = 16
_CH = 49920


def _flatten(tt, tail_flat):
    d, n_rows = tt.shape
    cols_per_w = d // _NW
    rag = n_rows % 128
    aligned = n_rows - rag
    n_full = aligned // _CH
    rem = aligned - n_full * _CH
    mesh = plsc.VectorSubcoreMesh(core_axis_name="c", subcore_axis_name="s")

    @functools.partial(
        pl.kernel,
        mesh=mesh,
        out_type=jax.ShapeDtypeStruct((d * n_rows,), jnp.float32),
        scratch_types=[
            pltpu.VMEM((_CH,), jnp.float32),
            pltpu.VMEM((_CH,), jnp.float32),
            pltpu.VMEM((d * rag,), jnp.float32),
            pltpu.SemaphoreType.DMA,
            pltpu.SemaphoreType.DMA,
        ],
    )
    def k(tt_hbm, tail_hbm, flat_hbm, buf0, buf1, tailv, s0, s1):
        wid = lax.axis_index("s") * 2 + lax.axis_index("c")
        bufs = (buf0, buf1)
        sems = (s0, s1)
        pending = {}
        step = 0
        for j in range(cols_per_w):
            c = wid * cols_per_w + j
            col = tt_hbm.at[c]
            cbase = c * n_rows
            for kk in range(n_full + 1):
                ln = _CH if kk < n_full else rem
                b = step % 2
                step += 1
                if b in pending:
                    pending[b].wait()
                src = col.at[pl.ds(pl.multiple_of(kk * _CH, 128), ln)]
                dst = flat_hbm.at[pl.ds(pl.multiple_of(cbase + kk * _CH, 8), ln)]
                stage = bufs[b] if ln == _CH else bufs[b].at[pl.ds(0, ln)]
                pltpu.sync_copy(src, stage)
                pending[b] = pltpu.async_copy(stage, dst, sems[b])
        for b in pending:
            pending[b].wait()

        @pl.when(wid == 0)
        def _rag():
            pltpu.sync_copy(tail_hbm, tailv)
            for i in range(d):
                dst0 = pl.multiple_of(i * n_rows + aligned, 8)
                pltpu.sync_copy(
                    tailv.at[pl.ds(i * rag, rag)], flat_hbm.at[pl.ds(dst0, rag)]
                )

    return k(tt, tail_flat)


def _col_gather(flat, it, n_rows):
    d, b = it.shape
    cols_per_w = d // _NW
    mesh = plsc.VectorSubcoreMesh(core_axis_name="c", subcore_axis_name="s")

    @functools.partial(
        pl.kernel,
        mesh=mesh,
        out_type=jax.ShapeDtypeStruct((d, b), jnp.float32),
        scratch_types=[
            pltpu.VMEM((b,), jnp.int32),
            pltpu.VMEM((b,), jnp.float32),
            pltpu.SemaphoreType.DMA,
        ],
    )
    def k(flat_hbm, it_hbm, out_hbm, idx_v, out_v, sem):
        wid = lax.axis_index("s") * 2 + lax.axis_index("c")
        for j in range(cols_per_w):
            c = wid * cols_per_w + j
            pltpu.sync_copy(it_hbm.at[c], idx_v)
            col_base = c * n_rows

            def body(v, _):
                off = v * _L
                idx_v[pl.ds(off, _L)] = idx_v[pl.ds(off, _L)] + col_base
                return 0

            lax.fori_loop(0, b // _L, body, 0)
            pltpu.async_copy(flat_hbm.at[idx_v], out_v, sem).wait()
            pltpu.sync_copy(out_v, out_hbm.at[c])

    return k(flat, it)


def kernel(input_tensor, indices):
    n_rows, d = input_tensor.shape
    rag = n_rows % 128
    tail_flat = input_tensor[n_rows - rag :, :].T.reshape(-1)
    flat = _flatten(input_tensor.T, tail_flat)
    out_t = _col_gather(flat, indices.astype(jnp.int32).T, n_rows)
    return out_t.T
